# Initial kernel scaffold; baseline (speedup 1.0000x reference)
import jax
import jax.numpy as jnp
from jax.experimental import pallas as pl


def kernel(cumulative_positions, position_embeddings):
    raise NotImplementedError("write your pallas kernel here")


# speedup vs baseline: 2.4053x; 2.4053x over previous
import functools

import jax
import jax.numpy as jnp
from jax import lax
from jax.experimental import pallas as pl
from jax.experimental.pallas import tpu as pltpu
from jax.experimental.pallas import tpu_sc as plsc

D_MODEL = 128
NUM_WORKERS = 32
CHUNK = 512


def _sc_gather(idx_flat, table, n_total):
    n_per_w = n_total // NUM_WORKERS
    steps = n_per_w // CHUNK
    mesh = plsc.VectorSubcoreMesh(core_axis_name="c", subcore_axis_name="s")

    @functools.partial(
        pl.kernel,
        mesh=mesh,
        out_type=jax.ShapeDtypeStruct((n_total, D_MODEL), jnp.float32),
        scratch_types=[
            pltpu.VMEM((CHUNK,), jnp.int32),
            pltpu.VMEM((CHUNK, D_MODEL), jnp.float32),
            pltpu.SemaphoreType.DMA,
        ],
    )
    def k(idx_hbm, table_hbm, out_hbm, idx_v, rows_v, sem):
        wid = lax.axis_index("s") * 2 + lax.axis_index("c")
        base = wid * n_per_w

        def body(i, carry):
            off = base + i * CHUNK
            pltpu.sync_copy(idx_hbm.at[pl.ds(off, CHUNK)], idx_v)
            pltpu.async_copy(table_hbm.at[idx_v], rows_v, sem).wait()
            pltpu.sync_copy(rows_v, out_hbm.at[pl.ds(off, CHUNK)])
            return carry

        lax.fori_loop(0, steps, body, 0)

    return k(idx_flat, table)


def kernel(cumulative_positions, position_embeddings):
    b, s = cumulative_positions.shape
    n_total = b * s
    idx_flat = cumulative_positions.reshape(n_total).astype(jnp.int32)
    out = _sc_gather(idx_flat, position_embeddings, n_total)
    return out.reshape(b, s, D_MODEL)


# speedup vs baseline: 2.4226x; 1.0072x over previous
import functools

import jax
import jax.numpy as jnp
from jax import lax
from jax.experimental import pallas as pl
from jax.experimental.pallas import tpu as pltpu
from jax.experimental.pallas import tpu_sc as plsc

D_MODEL = 128
NUM_WORKERS = 32
CHUNK = 400
NBUF = 2


def _sc_gather(idx_flat, table, n_total):
    n_per_w = n_total // NUM_WORKERS
    steps = n_per_w // CHUNK
    mesh = plsc.VectorSubcoreMesh(core_axis_name="c", subcore_axis_name="s")

    @functools.partial(
        pl.kernel,
        mesh=mesh,
        out_type=jax.ShapeDtypeStruct((n_total, D_MODEL), jnp.float32),
        scratch_types=[
            pltpu.VMEM((n_per_w,), jnp.int32),
            pltpu.VMEM((CHUNK, D_MODEL), jnp.float32),
            pltpu.VMEM((CHUNK, D_MODEL), jnp.float32),
            pltpu.SemaphoreType.DMA,
            pltpu.SemaphoreType.DMA,
            pltpu.SemaphoreType.DMA,
            pltpu.SemaphoreType.DMA,
        ],
    )
    def k(idx_hbm, table_hbm, out_hbm, idx_v, rows0, rows1, g0, g1, w0, w1):
        wid = lax.axis_index("s") * 2 + lax.axis_index("c")
        base = wid * n_per_w
        pltpu.sync_copy(idx_hbm.at[pl.ds(base, n_per_w)], idx_v)

        rows = (rows0, rows1)
        gsem = (g0, g1)
        wsem = (w0, w1)

        def gather(i, b):
            return pltpu.make_async_copy(
                table_hbm.at[idx_v.at[pl.ds(i * CHUNK, CHUNK)]], rows[b], gsem[b]
            )

        def write(i, b):
            return pltpu.make_async_copy(
                rows[b], out_hbm.at[pl.ds(base + i * CHUNK, CHUNK)], wsem[b]
            )

        for b in range(NBUF):
            gather(b, b).start()

        def body(grp, carry):
            for b in range(NBUF):
                i = grp * NBUF + b
                gather(i, b).wait()
                write(i, b).start()
                write(i, b).wait()
                nxt = i + NBUF

                @pl.when(nxt < steps)
                def _():
                    gather(nxt, b).start()

            return carry

        lax.fori_loop(0, steps // NBUF, body, 0)

    return k(idx_flat, table)


def kernel(cumulative_positions, position_embeddings):
    b, s = cumulative_positions.shape
    n_total = b * s
    idx_flat = cumulative_positions.reshape(n_total).astype(jnp.int32)
    out = _sc_gather(idx_flat, position_embeddings, n_total)
    return out.reshape(b, s, D_MODEL)
